# baseline (speedup 1.0000x reference)
import functools

import jax
import jax.numpy as jnp
from jax import lax
from jax.experimental import pallas as pl
from jax.experimental.pallas import tpu as pltpu
from jax.experimental.pallas import tpu_sc as plsc

B = 16384
N = 2048
M = 2048
L = 16
NC = 2
NS = 16
NW = NC * NS
ROWS_PER_W = B // NW
RT = 8
NT = ROWS_PER_W // RT
COLS_PER_S = M // NS
NV = COLS_PER_S // L
RCH = 128
NCH = N // RCH


def _body(x_hbm, w_hbm, out_hbm,
          wb0, wb1, map_v, smap,
          inb0, inb1, outb0, outb1,
          semw0, semw1, semi0, semi1, semo0, semo1):
  s = lax.axis_index("s")
  c = lax.axis_index("c")
  c0 = s * COLS_PER_S

  wid = s * NC + c
  rbase = wid * ROWS_PER_W

  def x_slice(t):
    return x_hbm.at[pl.ds(rbase + t * RT, RT), :]

  wbufs = (wb0, wb1)
  semws = (semw0, semw1)

  def w_slice(ch):
    return w_hbm.at[pl.ds(ch * RCH, RCH), pl.ds(c0, COLS_PER_S)]

  pltpu.async_copy(w_slice(0), wb0, semw0)
  pltpu.async_copy(x_slice(0), inb0, semi0)
  pltpu.async_copy(x_slice(1), inb1, semi1)

  carry = tuple(
      [jnp.full((L,), -jnp.inf, jnp.float32) for _ in range(NV)]
      + [jnp.zeros((L,), jnp.int32) for _ in range(NV)])

  for ch in range(NCH):
    b = ch % 2
    pltpu.make_async_copy(w_slice(ch), wbufs[b], semws[b]).wait()
    if ch + 1 < NCH:
      pltpu.async_copy(w_slice(ch + 1), wbufs[1 - b], semws[1 - b])
    wb = wbufs[b]

    @plsc.parallel_loop(0, RCH, unroll=1, carry=carry)
    def rbody(r, cry, ch=ch, wb=wb):
      row = ch * RCH + r
      rowv = jnp.full((L,), row, dtype=jnp.int32)
      out = list(cry)
      for v in range(NV):
        w = wb[r, pl.ds(v * L, L)]
        gt = w > out[v]
        out[v] = jnp.where(gt, w, out[v])
        out[NV + v] = jnp.where(gt, rowv, out[NV + v])
      return tuple(out)

    carry = rbody

  for v in range(NV):
    map_v[pl.ds(c0 + v * L, L)] = carry[NV + v]

  pltpu.sync_copy(map_v.at[pl.ds(c0, COLS_PER_S)],
                  smap.at[pl.ds(c0, COLS_PER_S)])
  plsc.subcore_barrier()
  pltpu.sync_copy(smap, map_v)

  inbufs = (inb0, inb1)
  outbufs = (outb0, outb1)
  semis = (semi0, semi1)
  semos = (semo0, semo1)

  def o_slice(t):
    return out_hbm.at[pl.ds(rbase + t * RT, RT), :]

  def tile_pair(tt, _):
    for b in range(2):
      t = tt * 2 + b
      pltpu.make_async_copy(x_slice(t), inbufs[b], semis[b]).wait()

      @pl.when(tt > 0)
      def _wait_out():
        pltpu.make_async_copy(outbufs[b], o_slice(t - 2), semos[b]).wait()

      @plsc.parallel_loop(0, M // L, unroll=16)
      def jbody(jc, b=b):
        idx = map_v[pl.ds(jc * L, L)]
        for r in range(RT):
          rv = jnp.full((L,), r, dtype=jnp.int32)
          g = plsc.load_gather(inbufs[b], [rv, idx])
          outbufs[b][r, pl.ds(jc * L, L)] = g

      pltpu.async_copy(outbufs[b], o_slice(t), semos[b])

      @pl.when(t + 2 < NT)
      def _prefetch():
        pltpu.async_copy(x_slice(t + 2), inbufs[b], semis[b])
    return 0

  lax.fori_loop(0, NT // 2, tile_pair, 0)

  pltpu.make_async_copy(outbufs[0], o_slice(NT - 2), semos[0]).wait()
  pltpu.make_async_copy(outbufs[1], o_slice(NT - 1), semos[1]).wait()


@functools.partial(
    pl.kernel,
    mesh=plsc.VectorSubcoreMesh(core_axis_name="c", subcore_axis_name="s"),
    out_type=jax.ShapeDtypeStruct((B, M), jnp.float32),
    compiler_params=pltpu.CompilerParams(needs_layout_passes=False),
    scratch_types=[
        pltpu.VMEM((RCH, COLS_PER_S), jnp.float32),
        pltpu.VMEM((RCH, COLS_PER_S), jnp.float32),
        pltpu.VMEM((M,), jnp.int32),
        pltpu.VMEM_SHARED((M,), jnp.int32),
        pltpu.VMEM((RT, N), jnp.float32),
        pltpu.VMEM((RT, N), jnp.float32),
        pltpu.VMEM((RT, M), jnp.float32),
        pltpu.VMEM((RT, M), jnp.float32),
        pltpu.SemaphoreType.DMA,
        pltpu.SemaphoreType.DMA,
        pltpu.SemaphoreType.DMA,
        pltpu.SemaphoreType.DMA,
        pltpu.SemaphoreType.DMA,
        pltpu.SemaphoreType.DMA,
    ],
)
def _sc_gather(x_hbm, w_hbm, out_hbm,
               wb0, wb1, map_v, smap,
               inb0, inb1, outb0, outb1,
               semw0, semw1, semi0, semi1, semo0, semo1):
  _body(x_hbm, w_hbm, out_hbm,
        wb0, wb1, map_v, smap,
        inb0, inb1, outb0, outb1,
        semw0, semw1, semi0, semi1, semo0, semo1)


def kernel(x, weights, tau):
  del tau
  return _sc_gather(x, weights)


# speedup vs baseline: 1.0418x; 1.0418x over previous
import functools

import jax
import jax.numpy as jnp
from jax import lax
from jax.experimental import pallas as pl
from jax.experimental.pallas import tpu as pltpu
from jax.experimental.pallas import tpu_sc as plsc

B = 16384
N = 2048
M = 2048
L = 16
NC = 2
NS = 16
NW = NC * NS
ROWS_PER_W = B // NW
RT = 8
NT = ROWS_PER_W // RT
COLS_PER_S = M // NS
NV = COLS_PER_S // L
RCH = 128
NCH = N // RCH


def _body(x_hbm, w_hbm, out_hbm,
          wb0, wb1, map_v, smap,
          inb0, inb1, outb0, outb1,
          semw0, semw1, semi0, semi1, semo0, semo1):
  s = lax.axis_index("s")
  c = lax.axis_index("c")
  c0 = s * COLS_PER_S

  wid = s * NC + c
  rbase = wid * ROWS_PER_W

  def x_slice(t):
    return x_hbm.at[pl.ds(rbase + t * RT, RT), :]

  wbufs = (wb0, wb1)
  semws = (semw0, semw1)

  def w_slice(ch):
    return w_hbm.at[pl.ds(ch * RCH, RCH), pl.ds(c0, COLS_PER_S)]

  pltpu.async_copy(w_slice(0), wb0, semw0)

  carry = tuple(
      [jnp.full((L,), -jnp.inf, jnp.float32) for _ in range(NV)]
      + [jnp.zeros((L,), jnp.int32) for _ in range(NV)])

  for ch in range(NCH):
    b = ch % 2
    pltpu.make_async_copy(w_slice(ch), wbufs[b], semws[b]).wait()
    if ch + 1 < NCH:
      pltpu.async_copy(w_slice(ch + 1), wbufs[1 - b], semws[1 - b])
    wb = wbufs[b]

    @plsc.parallel_loop(0, RCH, unroll=1, carry=carry)
    def rbody(r, cry, ch=ch, wb=wb):
      row = ch * RCH + r
      rowv = jnp.full((L,), row, dtype=jnp.int32)
      out = list(cry)
      for v in range(NV):
        w = wb[r, pl.ds(v * L, L)]
        gt = w > out[v]
        out[v] = jnp.where(gt, w, out[v])
        out[NV + v] = jnp.where(gt, rowv, out[NV + v])
      return tuple(out)

    carry = rbody

  for v in range(NV):
    map_v[pl.ds(c0 + v * L, L)] = carry[NV + v]

  pltpu.sync_copy(map_v.at[pl.ds(c0, COLS_PER_S)],
                  smap.at[pl.ds(c0, COLS_PER_S)])
  plsc.subcore_barrier()
  pltpu.sync_copy(smap, map_v)

  inbufs = (inb0, inb1)
  outbufs = (outb0, outb1)
  semis = (semi0, semi1)
  semos = (semo0, semo1)

  def o_slice(t):
    return out_hbm.at[pl.ds(rbase + t * RT, RT), :]

  pltpu.async_copy(x_slice(0), inb0, semi0)
  pltpu.async_copy(x_slice(1), inb1, semi1)

  def tile_pair(tt, _):
    for b in range(2):
      t = tt * 2 + b
      pltpu.make_async_copy(x_slice(t), inbufs[b], semis[b]).wait()

      @pl.when(tt > 0)
      def _wait_out():
        pltpu.make_async_copy(outbufs[b], o_slice(t - 2), semos[b]).wait()

      @plsc.parallel_loop(0, M // L, unroll=8)
      def jbody(jc, b=b):
        idx = map_v[pl.ds(jc * L, L)]
        for r in range(RT):
          rv = jnp.full((L,), r, dtype=jnp.int32)
          g = plsc.load_gather(inbufs[b], [rv, idx])
          outbufs[b][r, pl.ds(jc * L, L)] = g

      pltpu.async_copy(outbufs[b], o_slice(t), semos[b])

      @pl.when(t + 2 < NT)
      def _prefetch():
        pltpu.async_copy(x_slice(t + 2), inbufs[b], semis[b])
    return 0

  lax.fori_loop(0, NT // 2, tile_pair, 0)

  pltpu.make_async_copy(outbufs[0], o_slice(NT - 2), semos[0]).wait()
  pltpu.make_async_copy(outbufs[1], o_slice(NT - 1), semos[1]).wait()


@functools.partial(
    pl.kernel,
    mesh=plsc.VectorSubcoreMesh(core_axis_name="c", subcore_axis_name="s"),
    out_type=jax.ShapeDtypeStruct((B, M), jnp.float32),
    compiler_params=pltpu.CompilerParams(needs_layout_passes=False),
    scratch_types=[
        pltpu.VMEM((RCH, COLS_PER_S), jnp.float32),
        pltpu.VMEM((RCH, COLS_PER_S), jnp.float32),
        pltpu.VMEM((M,), jnp.int32),
        pltpu.VMEM_SHARED((M,), jnp.int32),
        pltpu.VMEM((RT, N), jnp.float32),
        pltpu.VMEM((RT, N), jnp.float32),
        pltpu.VMEM((RT, M), jnp.float32),
        pltpu.VMEM((RT, M), jnp.float32),
        pltpu.SemaphoreType.DMA,
        pltpu.SemaphoreType.DMA,
        pltpu.SemaphoreType.DMA,
        pltpu.SemaphoreType.DMA,
        pltpu.SemaphoreType.DMA,
        pltpu.SemaphoreType.DMA,
    ],
)
def _sc_gather(x_hbm, w_hbm, out_hbm,
               wb0, wb1, map_v, smap,
               inb0, inb1, outb0, outb1,
               semw0, semw1, semi0, semi1, semo0, semo1):
  _body(x_hbm, w_hbm, out_hbm,
        wb0, wb1, map_v, smap,
        inb0, inb1, outb0, outb1,
        semw0, semw1, semi0, semi1, semo0, semo1)


def kernel(x, weights, tau):
  del tau
  return _sc_gather(x, weights)


# speedup vs baseline: 1.0448x; 1.0028x over previous
import functools

import jax
import jax.numpy as jnp
from jax import lax
from jax.experimental import pallas as pl
from jax.experimental.pallas import tpu as pltpu
from jax.experimental.pallas import tpu_sc as plsc

B = 16384
N = 2048
M = 2048
L = 16
NC = 2
NS = 16
NW = NC * NS
ROWS_PER_W = B // NW
RT = 8
NT = ROWS_PER_W // RT
COLS_PER_S = M // NS
NV = COLS_PER_S // L
RCH = 128
NCH = N // RCH


def _body(x_hbm, w_hbm, out_hbm,
          wb0, wb1, map_v, smap,
          inb0, inb1, outb0, outb1,
          semw0, semw1, semi0, semi1, semo0, semo1):
  s = lax.axis_index("s")
  c = lax.axis_index("c")
  c0 = s * COLS_PER_S

  wid = s * NC + c
  rbase = wid * ROWS_PER_W

  def x_slice(t):
    return x_hbm.at[pl.ds(rbase + t * RT, RT), :]

  wbufs = (wb0, wb1)
  semws = (semw0, semw1)

  def w_slice(ch):
    return w_hbm.at[pl.ds(ch * RCH, RCH), pl.ds(c0, COLS_PER_S)]

  pltpu.async_copy(w_slice(0), wb0, semw0)
  pltpu.async_copy(x_slice(0), inb0, semi0)
  pltpu.async_copy(x_slice(1), inb1, semi1)

  carry = tuple(
      [jnp.full((L,), -jnp.inf, jnp.float32) for _ in range(NV)]
      + [jnp.zeros((L,), jnp.int32) for _ in range(NV)])

  for ch in range(NCH):
    b = ch % 2
    pltpu.make_async_copy(w_slice(ch), wbufs[b], semws[b]).wait()
    if ch + 1 < NCH:
      pltpu.async_copy(w_slice(ch + 1), wbufs[1 - b], semws[1 - b])
    wb = wbufs[b]

    @plsc.parallel_loop(0, RCH, unroll=1, carry=carry)
    def rbody(r, cry, ch=ch, wb=wb):
      row = ch * RCH + r
      rowv = jnp.full((L,), row, dtype=jnp.int32)
      out = list(cry)
      for v in range(NV):
        w = wb[r, pl.ds(v * L, L)]
        gt = w > out[v]
        out[v] = jnp.where(gt, w, out[v])
        out[NV + v] = jnp.where(gt, rowv, out[NV + v])
      return tuple(out)

    carry = rbody

  for v in range(NV):
    map_v[pl.ds(c0 + v * L, L)] = carry[NV + v]

  pltpu.sync_copy(map_v.at[pl.ds(c0, COLS_PER_S)],
                  smap.at[pl.ds(c0, COLS_PER_S)])
  plsc.subcore_barrier()
  pltpu.sync_copy(smap, map_v)

  inbufs = (inb0, inb1)
  outbufs = (outb0, outb1)
  semis = (semi0, semi1)
  semos = (semo0, semo1)

  def o_slice(t):
    return out_hbm.at[pl.ds(rbase + t * RT, RT), :]

  def tile_pair(tt, _):
    for b in range(2):
      t = tt * 2 + b
      pltpu.make_async_copy(x_slice(t), inbufs[b], semis[b]).wait()

      @pl.when(tt > 0)
      def _wait_out():
        pltpu.make_async_copy(outbufs[b], o_slice(t - 2), semos[b]).wait()

      @plsc.parallel_loop(0, M // L, unroll=8)
      def jbody(jc, b=b):
        idx = map_v[pl.ds(jc * L, L)]
        for r in range(RT):
          rv = jnp.full((L,), r, dtype=jnp.int32)
          g = plsc.load_gather(inbufs[b], [rv, idx])
          outbufs[b][r, pl.ds(jc * L, L)] = g

      pltpu.async_copy(outbufs[b], o_slice(t), semos[b])

      @pl.when(t + 2 < NT)
      def _prefetch():
        pltpu.async_copy(x_slice(t + 2), inbufs[b], semis[b])
    return 0

  lax.fori_loop(0, NT // 2, tile_pair, 0)

  pltpu.make_async_copy(outbufs[0], o_slice(NT - 2), semos[0]).wait()
  pltpu.make_async_copy(outbufs[1], o_slice(NT - 1), semos[1]).wait()


@functools.partial(
    pl.kernel,
    mesh=plsc.VectorSubcoreMesh(core_axis_name="c", subcore_axis_name="s"),
    out_type=jax.ShapeDtypeStruct((B, M), jnp.float32),
    compiler_params=pltpu.CompilerParams(needs_layout_passes=False),
    scratch_types=[
        pltpu.VMEM((RCH, COLS_PER_S), jnp.float32),
        pltpu.VMEM((RCH, COLS_PER_S), jnp.float32),
        pltpu.VMEM((M,), jnp.int32),
        pltpu.VMEM_SHARED((M,), jnp.int32),
        pltpu.VMEM((RT, N), jnp.float32),
        pltpu.VMEM((RT, N), jnp.float32),
        pltpu.VMEM((RT, M), jnp.float32),
        pltpu.VMEM((RT, M), jnp.float32),
        pltpu.SemaphoreType.DMA,
        pltpu.SemaphoreType.DMA,
        pltpu.SemaphoreType.DMA,
        pltpu.SemaphoreType.DMA,
        pltpu.SemaphoreType.DMA,
        pltpu.SemaphoreType.DMA,
    ],
)
def _sc_gather(x_hbm, w_hbm, out_hbm,
               wb0, wb1, map_v, smap,
               inb0, inb1, outb0, outb1,
               semw0, semw1, semi0, semi1, semo0, semo1):
  _body(x_hbm, w_hbm, out_hbm,
        wb0, wb1, map_v, smap,
        inb0, inb1, outb0, outb1,
        semw0, semw1, semi0, semi1, semo0, semo1)


def kernel(x, weights, tau):
  del tau
  return _sc_gather(x, weights)


# speedup vs baseline: 1.0805x; 1.0342x over previous
import functools

import jax
import jax.numpy as jnp
from jax import lax
from jax.experimental import pallas as pl
from jax.experimental.pallas import tpu as pltpu
from jax.experimental.pallas import tpu_sc as plsc

B = 16384
N = 2048
M = 2048
L = 16
NC = 2
NS = 16
NW = NC * NS
ROWS_PER_W = B // NW
RT = 8
NT = ROWS_PER_W // RT
COLS_PER_S = M // NS
NV = COLS_PER_S // L
ROWS_PER_S = N // NS
PR = 8
NP = ROWS_PER_S // PR
NG = M // L


def _body(x_hbm, w_hbm, out_hbm,
          wb0, wb1, map_v, smap,
          maxst, argst, smax_sp, sarg_sp, pmax_v, parg_v,
          inb0, inb1, outb0, outb1,
          semw0, semw1, semi0, semi1, semo0, semo1):
  s = lax.axis_index("s")
  c = lax.axis_index("c")
  c0 = s * COLS_PER_S
  wid = s * NC + c
  rbase = wid * ROWS_PER_W

  def x_slice(t):
    return x_hbm.at[pl.ds(rbase + t * RT, RT), :]

  r0 = s * ROWS_PER_S
  wbufs = (wb0, wb1, outb0, outb1)
  semws = (semw0, semw1, semo0, semo1)

  def w_slice(p):
    return w_hbm.at[pl.ds(r0 + p * PR, PR), :]

  for p in range(4):
    pltpu.async_copy(w_slice(p), wbufs[p], semws[p])
  pltpu.async_copy(x_slice(0), inb0, semi0)
  pltpu.async_copy(x_slice(1), inb1, semi1)

  for p in range(NP):
    b = p % 4
    pltpu.make_async_copy(w_slice(p), wbufs[b], semws[b]).wait()
    wb = wbufs[b]

    @plsc.parallel_loop(0, NG, unroll=2)
    def gbody(g, p=p, wb=wb):
      if p == 0:
        m = jnp.full((L,), -jnp.inf, jnp.float32)
        a = jnp.zeros((L,), jnp.int32)
      else:
        m = maxst[pl.ds(g * L, L)]
        a = argst[pl.ds(g * L, L)]
      for r in range(PR):
        rowv = jnp.full((L,), r0 + (p * PR + r), dtype=jnp.int32)
        w = wb[r, pl.ds(g * L, L)]
        gt = w > m
        m = jnp.where(gt, w, m)
        a = jnp.where(gt, rowv, a)
      maxst[pl.ds(g * L, L)] = m
      argst[pl.ds(g * L, L)] = a

    if p + 4 < NP:
      pltpu.async_copy(w_slice(p + 4), wbufs[b], semws[b])

  pltpu.sync_copy(maxst, smax_sp.at[s])
  pltpu.sync_copy(argst, sarg_sp.at[s])
  plsc.subcore_barrier()
  pltpu.sync_copy(smax_sp.at[:, pl.ds(c0, COLS_PER_S)], pmax_v)
  pltpu.sync_copy(sarg_sp.at[:, pl.ds(c0, COLS_PER_S)], parg_v)

  for v in range(NV):
    m = jnp.full((L,), -jnp.inf, jnp.float32)
    a = jnp.zeros((L,), jnp.int32)
    for part in range(NS):
      pm = pmax_v[part, pl.ds(v * L, L)]
      pa = parg_v[part, pl.ds(v * L, L)]
      gt = pm > m
      m = jnp.where(gt, pm, m)
      a = jnp.where(gt, pa, a)
    map_v[pl.ds(c0 + v * L, L)] = a

  pltpu.sync_copy(map_v.at[pl.ds(c0, COLS_PER_S)],
                  smap.at[pl.ds(c0, COLS_PER_S)])
  plsc.subcore_barrier()
  pltpu.sync_copy(smap, map_v)

  inbufs = (inb0, inb1)
  outbufs = (outb0, outb1)
  semis = (semi0, semi1)
  semos = (semo0, semo1)

  def o_slice(t):
    return out_hbm.at[pl.ds(rbase + t * RT, RT), :]

  def tile_pair(tt, _):
    for b in range(2):
      t = tt * 2 + b
      pltpu.make_async_copy(x_slice(t), inbufs[b], semis[b]).wait()

      @pl.when(tt > 0)
      def _wait_out():
        pltpu.make_async_copy(outbufs[b], o_slice(t - 2), semos[b]).wait()

      @plsc.parallel_loop(0, M // L, unroll=8)
      def jbody(jc, b=b):
        idx = map_v[pl.ds(jc * L, L)]
        for r in range(RT):
          rv = jnp.full((L,), r, dtype=jnp.int32)
          g = plsc.load_gather(inbufs[b], [rv, idx])
          outbufs[b][r, pl.ds(jc * L, L)] = g

      pltpu.async_copy(outbufs[b], o_slice(t), semos[b])

      @pl.when(t + 2 < NT)
      def _prefetch():
        pltpu.async_copy(x_slice(t + 2), inbufs[b], semis[b])
    return 0

  lax.fori_loop(0, NT // 2, tile_pair, 0)

  pltpu.make_async_copy(outbufs[0], o_slice(NT - 2), semos[0]).wait()
  pltpu.make_async_copy(outbufs[1], o_slice(NT - 1), semos[1]).wait()


@functools.partial(
    pl.kernel,
    mesh=plsc.VectorSubcoreMesh(core_axis_name="c", subcore_axis_name="s"),
    out_type=jax.ShapeDtypeStruct((B, M), jnp.float32),
    compiler_params=pltpu.CompilerParams(needs_layout_passes=False),
    scratch_types=[
        pltpu.VMEM((PR, N), jnp.float32),
        pltpu.VMEM((PR, N), jnp.float32),
        pltpu.VMEM((M,), jnp.int32),
        pltpu.VMEM_SHARED((M,), jnp.int32),
        pltpu.VMEM((M,), jnp.float32),
        pltpu.VMEM((M,), jnp.int32),
        pltpu.VMEM_SHARED((NS, M), jnp.float32),
        pltpu.VMEM_SHARED((NS, M), jnp.int32),
        pltpu.VMEM((NS, COLS_PER_S), jnp.float32),
        pltpu.VMEM((NS, COLS_PER_S), jnp.int32),
        pltpu.VMEM((RT, N), jnp.float32),
        pltpu.VMEM((RT, N), jnp.float32),
        pltpu.VMEM((RT, M), jnp.float32),
        pltpu.VMEM((RT, M), jnp.float32),
        pltpu.SemaphoreType.DMA,
        pltpu.SemaphoreType.DMA,
        pltpu.SemaphoreType.DMA,
        pltpu.SemaphoreType.DMA,
        pltpu.SemaphoreType.DMA,
        pltpu.SemaphoreType.DMA,
    ],
)
def _sc_gather(x_hbm, w_hbm, out_hbm,
               wb0, wb1, map_v, smap,
               maxst, argst, smax_sp, sarg_sp, pmax_v, parg_v,
               inb0, inb1, outb0, outb1,
               semw0, semw1, semi0, semi1, semo0, semo1):
  _body(x_hbm, w_hbm, out_hbm,
        wb0, wb1, map_v, smap,
        maxst, argst, smax_sp, sarg_sp, pmax_v, parg_v,
        inb0, inb1, outb0, outb1,
        semw0, semw1, semi0, semi1, semo0, semo1)


def kernel(x, weights, tau):
  del tau
  return _sc_gather(x, weights)
